# Initial kernel scaffold; baseline (speedup 1.0000x reference)
import jax
import jax.numpy as jnp
from jax.experimental import pallas as pl


def kernel(x, edge_index, Wfc0, wl0, wr0, Wfc1, wl1, wr1, Wres1, Wfc2, wl2, wr2, Wres2):
    raise NotImplementedError("write your pallas kernel here")


# speedup vs baseline: 11.2978x; 11.2978x over previous
import functools

import jax
import jax.numpy as jnp
import numpy as np
from jax.experimental import pallas as pl

N = 10000
E = 320000
HID = 8

_ROWS = 1000


def _prepare_body(x_ref, w_ref, wl_ref, wr_ref, s_ref, ft_ref, a1_ref, a2_ref):
    ft = jax.lax.dot(x_ref[...], w_ref[...].T,
                     precision=jax.lax.Precision.DEFAULT,
                     preferred_element_type=jnp.float32)
    ft_ref[...] = ft
    a1_ref[...] = jax.lax.dot(ft * wl_ref[...], s_ref[...],
                              precision=jax.lax.Precision.DEFAULT,
                              preferred_element_type=jnp.float32)
    a2_ref[...] = jax.lax.dot(ft * wr_ref[...], s_ref[...],
                              precision=jax.lax.Precision.DEFAULT,
                              preferred_element_type=jnp.float32)


def _prepare(last, Wfc, wl, wr):
    H, D, Din = Wfc.shape
    F = H * D
    Wstack = Wfc.reshape(F, Din)
    wlvec = wl.reshape(1, F)
    wrvec = wr.reshape(1, F)
    S = jnp.asarray(np.kron(np.eye(H), np.ones((D, 1))), dtype=jnp.float32)
    grid = (N // _ROWS,)
    ft, a1, a2 = pl.pallas_call(
        _prepare_body,
        grid=grid,
        in_specs=[
            pl.BlockSpec((_ROWS, Din), lambda i: (i, 0)),
            pl.BlockSpec((F, Din), lambda i: (0, 0)),
            pl.BlockSpec((1, F), lambda i: (0, 0)),
            pl.BlockSpec((1, F), lambda i: (0, 0)),
            pl.BlockSpec((F, H), lambda i: (0, 0)),
        ],
        out_specs=[
            pl.BlockSpec((_ROWS, F), lambda i: (i, 0)),
            pl.BlockSpec((_ROWS, H), lambda i: (i, 0)),
            pl.BlockSpec((_ROWS, H), lambda i: (i, 0)),
        ],
        out_shape=[
            jax.ShapeDtypeStruct((N, F), jnp.float32),
            jax.ShapeDtypeStruct((N, H), jnp.float32),
            jax.ShapeDtypeStruct((N, H), jnp.float32),
        ],
    )(last, Wstack, wlvec, wrvec, S)
    return ft, a1, a2


def _edge_phase(ft, a1, a2, src, dst, H, D):
    a2max = jnp.max(a2, axis=0)
    c = jax.nn.leaky_relu(a1 + a2max[None, :], negative_slope=0.01)
    e = jax.nn.leaky_relu(a1[dst] + a2[src], negative_slope=0.01)
    ee = jnp.exp(e - c[dst])
    den = jax.ops.segment_sum(ee, dst, num_segments=N)
    w = jnp.repeat(ee, D, axis=1)
    accum = jax.ops.segment_sum(w * ft[src], dst, num_segments=N)
    deninv = 1.0 / jnp.where(den > 0.0, den, 1.0)
    return accum * jnp.repeat(deninv, D, axis=1)


def _layer(last, Wfc, wl, wr, Wres, src, dst):
    H, D, Din = Wfc.shape
    ft, a1, a2 = _prepare(last, Wfc, wl, wr)
    accum = _edge_phase(ft, a1, a2, src, dst, H, D)
    if Wres is not None:
        accum = jax.lax.dot(last, Wres.reshape(H * D, Din).T,
                            precision=jax.lax.Precision.DEFAULT) + accum
    return jax.nn.elu(accum)


def kernel(x, edge_index, Wfc0, wl0, wr0, Wfc1, wl1, wr1, Wres1, Wfc2, wl2, wr2, Wres2):
    src = edge_index[0]
    dst = edge_index[1]
    last = _layer(x, Wfc0, wl0, wr0, None, src, dst)
    last = _layer(last, Wfc1, wl1, wr1, Wres1, src, dst)
    out = _layer(last, Wfc2, wl2, wr2, Wres2, src, dst)
    return out


# speedup vs baseline: 94.8176x; 8.3926x over previous
import functools

import jax
import jax.numpy as jnp
import numpy as np
from jax import lax
from jax.experimental import pallas as pl
from jax.experimental.pallas import tpu as pltpu
from jax.experimental.pallas import tpu_sc as plsc

N = 10000
E = 320000
F = 64
NT = 32
CH = 10000
_ROWS = 1000

_MESH = dict(core_axis_name="c", subcore_axis_name="s")


def _prepare_body(x_ref, w_ref, wl_ref, wr_ref, ft_ref, a1_ref, a2_ref):
    ft = jnp.dot(x_ref[...], w_ref[...].T, preferred_element_type=jnp.float32)
    ft_ref[...] = ft
    a1_ref[...] = jnp.dot(ft, wl_ref[...], preferred_element_type=jnp.float32)
    a2_ref[...] = jnp.dot(ft, wr_ref[...], preferred_element_type=jnp.float32)


def _blockdiag(w):
    H, _, D = w.shape
    mask = jnp.asarray(np.kron(np.eye(H), np.ones((D, 1))), dtype=jnp.float32)
    return w.reshape(1, H * D).T * mask


def _prepare(last, Wfc, wl, wr):
    H, D, Din = Wfc.shape
    ft, a1, a2 = pl.pallas_call(
        _prepare_body,
        grid=(N // _ROWS,),
        in_specs=[
            pl.BlockSpec((_ROWS, Din), lambda i: (i, 0)),
            pl.BlockSpec((F, Din), lambda i: (0, 0)),
            pl.BlockSpec((F, H), lambda i: (0, 0)),
            pl.BlockSpec((F, H), lambda i: (0, 0)),
        ],
        out_specs=[
            pl.BlockSpec((_ROWS, F), lambda i: (i, 0)),
            pl.BlockSpec((_ROWS, H), lambda i: (i, 0)),
            pl.BlockSpec((_ROWS, H), lambda i: (i, 0)),
        ],
        out_shape=[
            jax.ShapeDtypeStruct((N, F), jnp.float32),
            jax.ShapeDtypeStruct((N, H), jnp.float32),
            jax.ShapeDtypeStruct((N, H), jnp.float32),
        ],
    )(last, Wfc.reshape(F, Din), _blockdiag(wl), _blockdiag(wr))
    return ft, a1, a2


def _phase_a(H):
    Q = NT // H
    EQ = E // Q
    nchunk = EQ // CH

    @functools.partial(
        pl.kernel,
        out_type=[
            jax.ShapeDtypeStruct((H * E,), jnp.float32),
            jax.ShapeDtypeStruct((NT * N,), jnp.float32),
        ],
        mesh=plsc.VectorSubcoreMesh(**_MESH),
        compiler_params=pltpu.CompilerParams(needs_layout_passes=False),
        scratch_types=[
            pltpu.VMEM((N,), jnp.float32),
            pltpu.VMEM((N,), jnp.float32),
            pltpu.VMEM((N,), jnp.float32),
            pltpu.VMEM((N,), jnp.float32),
            pltpu.VMEM((CH,), jnp.int32),
            pltpu.VMEM((CH,), jnp.int32),
            pltpu.VMEM((CH,), jnp.float32),
        ],
    )
    def k(src_hbm, dst_hbm, a1t_hbm, a2t_hbm, ct_hbm, ee_hbm, denp_hbm,
          A1v, A2v, Cv, DENv, SRCv, DSTv, EEv):
        wid = lax.axis_index("s") * 2 + lax.axis_index("c")
        h = wid // Q
        q = wid % Q
        pltpu.sync_copy(a1t_hbm.at[h], A1v)
        pltpu.sync_copy(a2t_hbm.at[h], A2v)
        pltpu.sync_copy(ct_hbm.at[h], Cv)

        def zero(i, _):
            DENv[pl.ds(i * 16, 16)] = jnp.zeros((16,), jnp.float32)
            return 0
        lax.fori_loop(0, N // 16, zero, 0)

        def chunk(kk, _):
            off = q * EQ + kk * CH
            pltpu.sync_copy(src_hbm.at[pl.ds(off, CH)], SRCv)
            pltpu.sync_copy(dst_hbm.at[pl.ds(off, CH)], DSTv)

            def body(i, _):
                s16 = SRCv[pl.ds(i * 16, 16)]
                d16 = DSTv[pl.ds(i * 16, 16)]
                a1d = plsc.load_gather(A1v, [d16])
                a2s = plsc.load_gather(A2v, [s16])
                cd = plsc.load_gather(Cv, [d16])
                t = a1d + a2s
                ee = jnp.exp(jnp.maximum(t, t * 0.01) - cd)
                EEv[pl.ds(i * 16, 16)] = ee
                plsc.addupdate_scatter(DENv, [d16], ee)
                return 0
            lax.fori_loop(0, CH // 16, body, 0)
            pltpu.sync_copy(EEv, ee_hbm.at[pl.ds(h * E + off, CH)])
            return 0
        lax.fori_loop(0, nchunk, chunk, 0)
        pltpu.sync_copy(DENv, denp_hbm.at[pl.ds(wid * N, N)])

    return k


def _denprep_body(H, denp_ref, dinv_ref):
    den = jnp.sum(denp_ref[...].reshape(H, NT // H, N), axis=1)
    dinv_ref[...] = 1.0 / jnp.where(den > 0.0, den, 1.0)


def _denprep(denp, H):
    return pl.pallas_call(
        functools.partial(_denprep_body, H),
        out_shape=jax.ShapeDtypeStruct((H, N), jnp.float32),
    )(denp)


def _phase_b(H):
    nchunk = E // CH

    @functools.partial(
        pl.kernel,
        out_type=jax.ShapeDtypeStruct((F, N), jnp.float32),
        mesh=plsc.VectorSubcoreMesh(**_MESH),
        compiler_params=pltpu.CompilerParams(needs_layout_passes=False),
        scratch_types=[
            pltpu.VMEM((N,), jnp.float32),
            pltpu.VMEM((N,), jnp.float32),
            pltpu.VMEM((N,), jnp.float32),
            pltpu.VMEM((N,), jnp.float32),
            pltpu.VMEM((N,), jnp.float32),
            pltpu.VMEM((CH,), jnp.int32),
            pltpu.VMEM((CH,), jnp.int32),
            pltpu.VMEM((CH,), jnp.float32),
        ],
    )
    def k(src_hbm, dst_hbm, ftt_hbm, ee_hbm, dinv_hbm, acct_hbm,
          FTa, FTb, ACCa, ACCb, DIv, SRCv, DSTv, EEv):
        wid = lax.axis_index("s") * 2 + lax.axis_index("c")
        h = wid // (NT // H)
        pltpu.sync_copy(ftt_hbm.at[2 * wid], FTa)
        pltpu.sync_copy(ftt_hbm.at[2 * wid + 1], FTb)
        pltpu.sync_copy(dinv_hbm.at[h], DIv)

        def zero(i, _):
            z = jnp.zeros((16,), jnp.float32)
            ACCa[pl.ds(i * 16, 16)] = z
            ACCb[pl.ds(i * 16, 16)] = z
            return 0
        lax.fori_loop(0, N // 16, zero, 0)

        def chunk(kk, _):
            off = kk * CH
            pltpu.sync_copy(src_hbm.at[pl.ds(off, CH)], SRCv)
            pltpu.sync_copy(dst_hbm.at[pl.ds(off, CH)], DSTv)
            pltpu.sync_copy(ee_hbm.at[pl.ds(h * E + off, CH)], EEv)

            def body(i, _):
                s16 = SRCv[pl.ds(i * 16, 16)]
                d16 = DSTv[pl.ds(i * 16, 16)]
                w16 = EEv[pl.ds(i * 16, 16)]
                fa = plsc.load_gather(FTa, [s16])
                plsc.addupdate_scatter(ACCa, [d16], w16 * fa)
                fb = plsc.load_gather(FTb, [s16])
                plsc.addupdate_scatter(ACCb, [d16], w16 * fb)
                return 0
            lax.fori_loop(0, CH // 16, body, 0)
            return 0
        lax.fori_loop(0, nchunk, chunk, 0)

        def norm(i, _):
            ds_ = pl.ds(i * 16, 16)
            dv = DIv[ds_]
            ACCa[ds_] = ACCa[ds_] * dv
            ACCb[ds_] = ACCb[ds_] * dv
            return 0
        lax.fori_loop(0, N // 16, norm, 0)
        pltpu.sync_copy(ACCa, acct_hbm.at[2 * wid])
        pltpu.sync_copy(ACCb, acct_hbm.at[2 * wid + 1])

    return k


def _finalize_res_body(acc_ref, last_ref, wres_ref, out_ref):
    v = acc_ref[...] + jnp.dot(last_ref[...], wres_ref[...].T,
                               preferred_element_type=jnp.float32)
    out_ref[...] = jnp.where(v > 0.0, v, jnp.exp(v) - 1.0)


def _finalize_nores_body(acc_ref, out_ref):
    v = acc_ref[...]
    out_ref[...] = jnp.where(v > 0.0, v, jnp.exp(v) - 1.0)


def _finalize(accum, last, Wres):
    if Wres is None:
        return pl.pallas_call(
            _finalize_nores_body,
            grid=(N // _ROWS,),
            in_specs=[pl.BlockSpec((_ROWS, F), lambda i: (i, 0))],
            out_specs=pl.BlockSpec((_ROWS, F), lambda i: (i, 0)),
            out_shape=jax.ShapeDtypeStruct((N, F), jnp.float32),
        )(accum)
    H, D, Din = Wres.shape
    return pl.pallas_call(
        _finalize_res_body,
        grid=(N // _ROWS,),
        in_specs=[
            pl.BlockSpec((_ROWS, F), lambda i: (i, 0)),
            pl.BlockSpec((_ROWS, Din), lambda i: (i, 0)),
            pl.BlockSpec((F, Din), lambda i: (0, 0)),
        ],
        out_specs=pl.BlockSpec((_ROWS, F), lambda i: (i, 0)),
        out_shape=jax.ShapeDtypeStruct((N, F), jnp.float32),
    )(accum, last, Wres.reshape(F, Din))


def _layer(last, Wfc, wl, wr, Wres, src, dst):
    H, D, Din = Wfc.shape
    ft, a1, a2 = _prepare(last, Wfc, wl, wr)
    a2max = jnp.max(a2, axis=0)
    cpre = jax.nn.leaky_relu(a1 + a2max[None, :], negative_slope=0.01)
    a1t = a1.T
    a2t = a2.T
    ct = cpre.T
    ee, denp = _phase_a(H)(src, dst, a1t, a2t, ct)
    dinv = _denprep(denp.reshape(NT, N), H)
    acct = _phase_b(H)(src, dst, ft.T, ee, dinv)
    return _finalize(acct.T, last, Wres)


def kernel(x, edge_index, Wfc0, wl0, wr0, Wfc1, wl1, wr1, Wres1, Wfc2, wl2, wr2, Wres2):
    src = edge_index[0]
    dst = edge_index[1]
    last = _layer(x, Wfc0, wl0, wr0, None, src, dst)
    last = _layer(last, Wfc1, wl1, wr1, Wres1, src, dst)
    out = _layer(last, Wfc2, wl2, wr2, Wres2, src, dst)
    return out


# speedup vs baseline: 115.7719x; 1.2210x over previous
import functools

import jax
import jax.numpy as jnp
import numpy as np
from jax import lax
from jax.experimental import pallas as pl
from jax.experimental.pallas import tpu as pltpu
from jax.experimental.pallas import tpu_sc as plsc

N = 10000
E = 320000
F = 64
NT = 32
CH = 10000
_ROWS = 1000

_MESH = dict(core_axis_name="c", subcore_axis_name="s")


def _prepare_body(x_ref, w_ref, wl_ref, wr_ref, ft_ref, a1_ref, a2_ref):
    ft = jnp.dot(x_ref[...], w_ref[...].T, preferred_element_type=jnp.float32)
    ft_ref[...] = ft
    a1_ref[...] = jnp.dot(ft, wl_ref[...], preferred_element_type=jnp.float32)
    a2_ref[...] = jnp.dot(ft, wr_ref[...], preferred_element_type=jnp.float32)


def _blockdiag(w):
    H, _, D = w.shape
    mask = jnp.asarray(np.kron(np.eye(H), np.ones((D, 1))), dtype=jnp.float32)
    return w.reshape(1, H * D).T * mask


def _prepare(last, Wfc, wl, wr):
    H, D, Din = Wfc.shape
    ft, a1, a2 = pl.pallas_call(
        _prepare_body,
        grid=(N // _ROWS,),
        in_specs=[
            pl.BlockSpec((_ROWS, Din), lambda i: (i, 0)),
            pl.BlockSpec((F, Din), lambda i: (0, 0)),
            pl.BlockSpec((F, H), lambda i: (0, 0)),
            pl.BlockSpec((F, H), lambda i: (0, 0)),
        ],
        out_specs=[
            pl.BlockSpec((_ROWS, F), lambda i: (i, 0)),
            pl.BlockSpec((_ROWS, H), lambda i: (i, 0)),
            pl.BlockSpec((_ROWS, H), lambda i: (i, 0)),
        ],
        out_shape=[
            jax.ShapeDtypeStruct((N, F), jnp.float32),
            jax.ShapeDtypeStruct((N, H), jnp.float32),
            jax.ShapeDtypeStruct((N, H), jnp.float32),
        ],
    )(last, Wfc.reshape(F, Din), _blockdiag(wl), _blockdiag(wr))
    return ft, a1, a2


_UNROLL = 5


def _phase_a(H):
    Q = NT // H
    EQ = E // Q
    nchunk = EQ // CH
    G = 16 * _UNROLL

    @functools.partial(
        pl.kernel,
        out_type=[
            jax.ShapeDtypeStruct((H * E,), jnp.float32),
            jax.ShapeDtypeStruct((NT * N,), jnp.float32),
        ],
        mesh=plsc.VectorSubcoreMesh(**_MESH),
        compiler_params=pltpu.CompilerParams(needs_layout_passes=False),
        scratch_types=[
            pltpu.VMEM((N,), jnp.float32),
            pltpu.VMEM((N,), jnp.float32),
            pltpu.VMEM((N,), jnp.float32),
            pltpu.VMEM((N,), jnp.float32),
            [pltpu.VMEM((CH,), jnp.int32)] * 2,
            [pltpu.VMEM((CH,), jnp.int32)] * 2,
            [pltpu.VMEM((CH,), jnp.float32)] * 2,
            [pltpu.SemaphoreType.DMA] * 2,
            [pltpu.SemaphoreType.DMA] * 2,
        ],
    )
    def k(src_hbm, dst_hbm, a1t_hbm, a2t_hbm, ct_hbm, ee_hbm, denp_hbm,
          A1v, A2v, Cv, DENv, SRCs, DSTs, EEs, semi, semo):
        wid = lax.axis_index("s") * 2 + lax.axis_index("c")
        h = wid // Q
        q = wid % Q
        pltpu.sync_copy(a1t_hbm.at[h], A1v)
        pltpu.sync_copy(a2t_hbm.at[h], A2v)
        pltpu.sync_copy(ct_hbm.at[h], Cv)

        def zero(i, _):
            DENv[pl.ds(i * 16, 16)] = jnp.zeros((16,), jnp.float32)
            return 0
        lax.fori_loop(0, N // 16, zero, 0)

        def start_in(kk, b):
            off = q * EQ + kk * CH
            pltpu.make_async_copy(src_hbm.at[pl.ds(off, CH)], SRCs[b], semi[b]).start()
            pltpu.make_async_copy(dst_hbm.at[pl.ds(off, CH)], DSTs[b], semi[b]).start()

        def wait_in(b):
            pltpu.make_async_copy(src_hbm.at[pl.ds(0, CH)], SRCs[b], semi[b]).wait()
            pltpu.make_async_copy(dst_hbm.at[pl.ds(0, CH)], DSTs[b], semi[b]).wait()

        def wait_out(b):
            pltpu.make_async_copy(EEs[b], ee_hbm.at[pl.ds(0, CH)], semo[b]).wait()

        def compute(kk, b):
            SRCv, DSTv, EEv = SRCs[b], DSTs[b], EEs[b]

            def body(i, _):
                for u in range(_UNROLL):
                    ds_ = pl.ds(i * G + u * 16, 16)
                    s16 = SRCv[ds_]
                    d16 = DSTv[ds_]
                    a1d = plsc.load_gather(A1v, [d16])
                    a2s = plsc.load_gather(A2v, [s16])
                    cd = plsc.load_gather(Cv, [d16])
                    t = a1d + a2s
                    ee = jnp.exp(jnp.maximum(t, t * 0.01) - cd)
                    EEv[ds_] = ee
                    plsc.addupdate_scatter(DENv, [d16], ee)
                return 0
            lax.fori_loop(0, CH // G, body, 0)
            off = q * EQ + kk * CH
            pltpu.make_async_copy(EEv, ee_hbm.at[pl.ds(h * E + off, CH)], semo[b]).start()

        if nchunk >= 2:
            start_in(0, 0)

            def pair(pp, _):
                base = 2 * pp

                @pl.when(base + 1 < nchunk)
                def _():
                    start_in(base + 1, 1)
                wait_in(0)

                @pl.when(pp > 0)
                def _():
                    wait_out(0)
                compute(base, 0)

                @pl.when(base + 2 < nchunk)
                def _():
                    start_in(base + 2, 0)
                wait_in(1)

                @pl.when(pp > 0)
                def _():
                    wait_out(1)
                compute(base + 1, 1)
                return 0
            lax.fori_loop(0, nchunk // 2, pair, 0)
            wait_out(0)
            wait_out(1)
        else:
            start_in(0, 0)
            wait_in(0)
            compute(0, 0)
            wait_out(0)
        pltpu.sync_copy(DENv, denp_hbm.at[pl.ds(wid * N, N)])

    return k


def _denprep_body(H, denp_ref, dinv_ref):
    den = jnp.sum(denp_ref[...].reshape(H, NT // H, N), axis=1)
    dinv_ref[...] = 1.0 / jnp.where(den > 0.0, den, 1.0)


def _denprep(denp, H):
    return pl.pallas_call(
        functools.partial(_denprep_body, H),
        out_shape=jax.ShapeDtypeStruct((H, N), jnp.float32),
    )(denp)


def _phase_b(H):
    nchunk = E // CH

    @functools.partial(
        pl.kernel,
        out_type=jax.ShapeDtypeStruct((F, N), jnp.float32),
        mesh=plsc.VectorSubcoreMesh(**_MESH),
        compiler_params=pltpu.CompilerParams(needs_layout_passes=False),
        scratch_types=[
            pltpu.VMEM((N,), jnp.float32),
            pltpu.VMEM((N,), jnp.float32),
            pltpu.VMEM((N,), jnp.float32),
            pltpu.VMEM((N,), jnp.float32),
            pltpu.VMEM((N,), jnp.float32),
            [pltpu.VMEM((CH,), jnp.int32)] * 2,
            [pltpu.VMEM((CH,), jnp.int32)] * 2,
            [pltpu.VMEM((CH,), jnp.float32)] * 2,
            [pltpu.SemaphoreType.DMA] * 2,
        ],
    )
    def k(src_hbm, dst_hbm, ftt_hbm, ee_hbm, dinv_hbm, acct_hbm,
          FTa, FTb, ACCa, ACCb, DIv, SRCs, DSTs, EEs, semi):
        wid = lax.axis_index("s") * 2 + lax.axis_index("c")
        h = wid // (NT // H)
        pltpu.sync_copy(ftt_hbm.at[2 * wid], FTa)
        pltpu.sync_copy(ftt_hbm.at[2 * wid + 1], FTb)
        pltpu.sync_copy(dinv_hbm.at[h], DIv)

        def zero(i, _):
            z = jnp.zeros((16,), jnp.float32)
            ACCa[pl.ds(i * 16, 16)] = z
            ACCb[pl.ds(i * 16, 16)] = z
            return 0
        lax.fori_loop(0, N // 16, zero, 0)

        G = 16 * _UNROLL

        def start_in(kk, b):
            off = kk * CH
            pltpu.make_async_copy(src_hbm.at[pl.ds(off, CH)], SRCs[b], semi[b]).start()
            pltpu.make_async_copy(dst_hbm.at[pl.ds(off, CH)], DSTs[b], semi[b]).start()
            pltpu.make_async_copy(ee_hbm.at[pl.ds(h * E + off, CH)], EEs[b], semi[b]).start()

        def wait_in(b):
            pltpu.make_async_copy(src_hbm.at[pl.ds(0, CH)], SRCs[b], semi[b]).wait()
            pltpu.make_async_copy(dst_hbm.at[pl.ds(0, CH)], DSTs[b], semi[b]).wait()
            pltpu.make_async_copy(ee_hbm.at[pl.ds(0, CH)], EEs[b], semi[b]).wait()

        def compute(b):
            SRCv, DSTv, EEv = SRCs[b], DSTs[b], EEs[b]

            def body(i, _):
                for u in range(_UNROLL):
                    ds_ = pl.ds(i * G + u * 16, 16)
                    s16 = SRCv[ds_]
                    d16 = DSTv[ds_]
                    w16 = EEv[ds_]
                    fa = plsc.load_gather(FTa, [s16])
                    plsc.addupdate_scatter(ACCa, [d16], w16 * fa)
                    fb = plsc.load_gather(FTb, [s16])
                    plsc.addupdate_scatter(ACCb, [d16], w16 * fb)
                return 0
            lax.fori_loop(0, CH // G, body, 0)

        start_in(0, 0)

        def pair(pp, _):
            base = 2 * pp

            @pl.when(base + 1 < nchunk)
            def _():
                start_in(base + 1, 1)
            wait_in(0)
            compute(0)

            @pl.when(base + 2 < nchunk)
            def _():
                start_in(base + 2, 0)
            wait_in(1)
            compute(1)
            return 0
        lax.fori_loop(0, nchunk // 2, pair, 0)

        def norm(i, _):
            ds_ = pl.ds(i * 16, 16)
            dv = DIv[ds_]
            ACCa[ds_] = ACCa[ds_] * dv
            ACCb[ds_] = ACCb[ds_] * dv
            return 0
        lax.fori_loop(0, N // 16, norm, 0)
        pltpu.sync_copy(ACCa, acct_hbm.at[2 * wid])
        pltpu.sync_copy(ACCb, acct_hbm.at[2 * wid + 1])

    return k


def _finalize_res_body(acc_ref, last_ref, wres_ref, out_ref):
    v = acc_ref[...] + jnp.dot(last_ref[...], wres_ref[...].T,
                               preferred_element_type=jnp.float32)
    out_ref[...] = jnp.where(v > 0.0, v, jnp.exp(v) - 1.0)


def _finalize_nores_body(acc_ref, out_ref):
    v = acc_ref[...]
    out_ref[...] = jnp.where(v > 0.0, v, jnp.exp(v) - 1.0)


def _finalize(accum, last, Wres):
    if Wres is None:
        return pl.pallas_call(
            _finalize_nores_body,
            grid=(N // _ROWS,),
            in_specs=[pl.BlockSpec((_ROWS, F), lambda i: (i, 0))],
            out_specs=pl.BlockSpec((_ROWS, F), lambda i: (i, 0)),
            out_shape=jax.ShapeDtypeStruct((N, F), jnp.float32),
        )(accum)
    H, D, Din = Wres.shape
    return pl.pallas_call(
        _finalize_res_body,
        grid=(N // _ROWS,),
        in_specs=[
            pl.BlockSpec((_ROWS, F), lambda i: (i, 0)),
            pl.BlockSpec((_ROWS, Din), lambda i: (i, 0)),
            pl.BlockSpec((F, Din), lambda i: (0, 0)),
        ],
        out_specs=pl.BlockSpec((_ROWS, F), lambda i: (i, 0)),
        out_shape=jax.ShapeDtypeStruct((N, F), jnp.float32),
    )(accum, last, Wres.reshape(F, Din))


def _layer(last, Wfc, wl, wr, Wres, src, dst):
    H, D, Din = Wfc.shape
    ft, a1, a2 = _prepare(last, Wfc, wl, wr)
    a2max = jnp.max(a2, axis=0)
    cpre = jax.nn.leaky_relu(a1 + a2max[None, :], negative_slope=0.01)
    a1t = a1.T
    a2t = a2.T
    ct = cpre.T
    ee, denp = _phase_a(H)(src, dst, a1t, a2t, ct)
    dinv = _denprep(denp.reshape(NT, N), H)
    acct = _phase_b(H)(src, dst, ft.T, ee, dinv)
    return _finalize(acct.T, last, Wres)


def kernel(x, edge_index, Wfc0, wl0, wr0, Wfc1, wl1, wr1, Wres1, Wfc2, wl2, wr2, Wres2):
    src = edge_index[0]
    dst = edge_index[1]
    last = _layer(x, Wfc0, wl0, wr0, None, src, dst)
    last = _layer(last, Wfc1, wl1, wr1, Wres1, src, dst)
    out = _layer(last, Wfc2, wl2, wr2, Wres2, src, dst)
    return out
